# baseline (speedup 1.0000x reference)
import functools

import jax
import jax.numpy as jnp
from jax import lax
from jax.experimental import pallas as pl
from jax.experimental.pallas import tpu as pltpu
from jax.experimental.pallas import tpu_sc as plsc

M = 24
OFFS = (0, 1, 5)
NUM_ITERS = 10
TEMP = 0.01
ALPHA = 100.0
NC, NS, LANES = 2, 16, 16
NW = NC * NS


def _tanh_alpha(x):
    t = jnp.exp((-2.0 * ALPHA) * jnp.abs(x))
    r = (1.0 - t) / (1.0 + t)
    return jnp.where(x >= 0, r, -r)


def _smin_pair(a, b):
    lo = jnp.minimum(a, b)
    hi = jnp.maximum(a, b)
    w = jnp.exp((lo - hi) * (1.0 / TEMP))
    return (lo + hi * w) / (1.0 + w)


def _bp_body(syn_hbm, prior_hbm, gamma_hbm, out_hbm,
             sv, v0, v1, v5, c0, c1, c5, lv, pr, gm):
    wid = lax.axis_index("s") * NC + lax.axis_index("c")
    bpw = sv.shape[1]
    nch = bpw // LANES

    pltpu.sync_copy(syn_hbm.at[wid], sv)
    pltpu.sync_copy(prior_hbm, pr)
    pltpu.sync_copy(gamma_hbm, gm)

    @pl.loop(0, nch)
    def _init(c):
        sl = pl.ds(c * LANES, LANES)
        for i in range(M):
            sv[i, sl] = 1.0 - 2.0 * sv[i, sl]
            lv[i, sl] = pr[i, :]
            v0[i, sl] = pr[i, :]
            v1[i, sl] = pr[(i + 1) % M, :]
            v5[i, sl] = pr[(i + 5) % M, :]

    @pl.loop(0, NUM_ITERS)
    def _iter(_):
        @plsc.parallel_loop(0, nch, unroll=2)
        def _chunk(c):
            sl = pl.ds(c * LANES, LANES)
            for i in range(M):
                s = sv[i, sl]
                m0, m1, m5 = v0[i, sl], v1[i, sl], v5[i, sl]
                t0, t1, t5 = _tanh_alpha(m0), _tanh_alpha(m1), _tanh_alpha(m5)
                a0, a1, a5 = jnp.abs(m0), jnp.abs(m1), jnp.abs(m5)
                c0[i, sl] = s * (t1 * t5) * _smin_pair(a1, a5)
                c1[i, sl] = s * (t0 * t5) * _smin_pair(a0, a5)
                c5[i, sl] = s * (t0 * t1) * _smin_pair(a0, a1)
            for j in range(M):
                inc = (c0[j, sl]
                       + c1[(j - 1) % M, sl]
                       + c5[(j - 5) % M, sl])
                pv = pr[j, :]
                gv = gm[j, :]
                lv[j, sl] = inc + (1.0 - gv) * pv + gv * lv[j, sl]
            for i in range(M):
                v0[i, sl] = lv[i, sl] - c0[i, sl]
                v1[i, sl] = lv[(i + 1) % M, sl] - c1[i, sl]
                v5[i, sl] = lv[(i + 5) % M, sl] - c5[i, sl]

    pltpu.sync_copy(lv, out_hbm.at[wid])


def _tc_body(syn_ref, prior_ref, gamma_ref, out_ref):
    p = prior_ref[...]
    g = gamma_ref[...]
    bb = syn_ref.shape[1]
    s = 1.0 - 2.0 * syn_ref[...]
    v0 = jnp.broadcast_to(p, (M, bb))
    v1 = jnp.broadcast_to(jnp.roll(p, -1, axis=0), (M, bb))
    v5 = jnp.broadcast_to(jnp.roll(p, -5, axis=0), (M, bb))
    lv = jnp.broadcast_to(p, (M, bb))
    for _ in range(NUM_ITERS):
        t0, t1, t5 = (jnp.tanh(ALPHA * v0), jnp.tanh(ALPHA * v1),
                      jnp.tanh(ALPHA * v5))
        a0, a1, a5 = jnp.abs(v0), jnp.abs(v1), jnp.abs(v5)
        c0 = s * (t1 * t5) * _smin_pair(a1, a5)
        c1 = s * (t0 * t5) * _smin_pair(a0, a5)
        c5 = s * (t0 * t1) * _smin_pair(a0, a1)
        inc = c0 + jnp.roll(c1, 1, axis=0) + jnp.roll(c5, 5, axis=0)
        lv = inc + (1.0 - g) * p + g * lv
        v0 = lv - c0
        v1 = jnp.roll(lv, -1, axis=0) - c1
        v5 = jnp.roll(lv, -5, axis=0) - c5
    out_ref[...] = lv


def _tc_forward(syn_t, prior_llr, gamma, block, interpret=False):
    batch_tc = syn_t.shape[1]
    assert batch_tc % block == 0, (batch_tc, block)
    return pl.pallas_call(
        _tc_body,
        grid=(batch_tc // block,),
        in_specs=[
            pl.BlockSpec((M, block), lambda i: (0, i)),
            pl.BlockSpec((M, 1), lambda i: (0, 0)),
            pl.BlockSpec((M, 1), lambda i: (0, 0)),
        ],
        out_specs=pl.BlockSpec((M, block), lambda i: (0, i)),
        out_shape=jax.ShapeDtypeStruct((M, batch_tc), jnp.float32),
        compiler_params=pltpu.CompilerParams(
            dimension_semantics=("parallel",)),
        interpret=interpret,
    )(syn_t, prior_llr.reshape(M, 1), gamma.reshape(M, 1))


def _sc_forward(syn_t, prior_llr, gamma):
    batch = syn_t.shape[1]
    assert batch % (NW * LANES) == 0, batch
    bpw = batch // NW

    syn_blocks = syn_t.reshape(M, NW, bpw).transpose(1, 0, 2)
    pr16 = jnp.broadcast_to(prior_llr[:, None], (M, LANES))
    gm16 = jnp.broadcast_to(gamma[:, None], (M, LANES))

    mesh = plsc.VectorSubcoreMesh(core_axis_name="c", subcore_axis_name="s")
    run = pl.kernel(
        _bp_body,
        out_type=jax.ShapeDtypeStruct((NW, M, bpw), jnp.float32),
        mesh=mesh,
        scratch_types=[pltpu.VMEM((M, bpw), jnp.float32)] * 8
                      + [pltpu.VMEM((M, LANES), jnp.float32)] * 2,
    )
    out_blocks = run(syn_blocks, pr16, gm16)
    return out_blocks.transpose(1, 0, 2).reshape(M, batch)


SC_BATCH = 2048
TC_BLOCK = 512


def kernel(syndromes, gamma, prior_llr):
    batch = syndromes.shape[0]
    syn_t = syndromes.astype(jnp.float32).T
    sc_b = SC_BATCH if batch > SC_BATCH else batch
    out_sc = _sc_forward(syn_t[:, :sc_b], prior_llr, gamma)
    if sc_b < batch:
        out_tc = _tc_forward(syn_t[:, sc_b:], prior_llr, gamma, TC_BLOCK)
        out = jnp.concatenate([out_sc, out_tc], axis=1)
    else:
        out = out_sc
    return out.T


# speedup vs baseline: 7.4075x; 7.4075x over previous
import functools

import jax
import jax.numpy as jnp
from jax import lax
from jax.experimental import pallas as pl
from jax.experimental.pallas import tpu as pltpu
from jax.experimental.pallas import tpu_sc as plsc

M = 24
OFFS = (0, 1, 5)
NUM_ITERS = 10
TEMP = 0.01
ALPHA = 100.0
NC, NS, LANES = 2, 16, 16
NW = NC * NS


def _tanh_alpha(x):
    t = jnp.exp((-2.0 * ALPHA) * jnp.abs(x))
    r = (1.0 - t) / (1.0 + t)
    return jnp.where(x >= 0, r, -r)


def _smin_pair(a, b):
    lo = jnp.minimum(a, b)
    hi = jnp.maximum(a, b)
    w = jnp.exp((lo - hi) * (1.0 / TEMP))
    return (lo + hi * w) / (1.0 + w)


def _bp_body(syn_hbm, prior_hbm, gamma_hbm, out_hbm,
             sv, v0, v1, v5, c0, c1, c5, lv, pr, gm):
    wid = lax.axis_index("s") * NC + lax.axis_index("c")
    bpw = sv.shape[1]
    nch = bpw // LANES

    pltpu.sync_copy(syn_hbm.at[wid], sv)
    pltpu.sync_copy(prior_hbm, pr)
    pltpu.sync_copy(gamma_hbm, gm)

    @pl.loop(0, nch)
    def _init(c):
        sl = pl.ds(c * LANES, LANES)
        for i in range(M):
            sv[i, sl] = 1.0 - 2.0 * sv[i, sl]
            lv[i, sl] = pr[i, :]
            v0[i, sl] = pr[i, :]
            v1[i, sl] = pr[(i + 1) % M, :]
            v5[i, sl] = pr[(i + 5) % M, :]

    @pl.loop(0, NUM_ITERS)
    def _iter(_):
        @pl.loop(0, nch)
        def _chunk(c):
            sl = pl.ds(c * LANES, LANES)
            for i in range(M):
                s = sv[i, sl]
                m0, m1, m5 = v0[i, sl], v1[i, sl], v5[i, sl]
                t0, t1, t5 = _tanh_alpha(m0), _tanh_alpha(m1), _tanh_alpha(m5)
                a0, a1, a5 = jnp.abs(m0), jnp.abs(m1), jnp.abs(m5)
                c0[i, sl] = s * (t1 * t5) * _smin_pair(a1, a5)
                c1[i, sl] = s * (t0 * t5) * _smin_pair(a0, a5)
                c5[i, sl] = s * (t0 * t1) * _smin_pair(a0, a1)
            for j in range(M):
                inc = (c0[j, sl]
                       + c1[(j - 1) % M, sl]
                       + c5[(j - 5) % M, sl])
                pv = pr[j, :]
                gv = gm[j, :]
                lv[j, sl] = inc + (1.0 - gv) * pv + gv * lv[j, sl]
            for i in range(M):
                v0[i, sl] = lv[i, sl] - c0[i, sl]
                v1[i, sl] = lv[(i + 1) % M, sl] - c1[i, sl]
                v5[i, sl] = lv[(i + 5) % M, sl] - c5[i, sl]

    pltpu.sync_copy(lv, out_hbm.at[wid])


def _tc_body(syn_ref, prior_ref, gamma_ref, out_ref):
    p = prior_ref[...]
    g = gamma_ref[...]
    bb = syn_ref.shape[1]
    s = 1.0 - 2.0 * syn_ref[...]
    v0 = jnp.broadcast_to(p, (M, bb))
    v1 = jnp.broadcast_to(jnp.roll(p, -1, axis=0), (M, bb))
    v5 = jnp.broadcast_to(jnp.roll(p, -5, axis=0), (M, bb))
    lv = jnp.broadcast_to(p, (M, bb))
    for _ in range(NUM_ITERS):
        t0, t1, t5 = (jnp.tanh(ALPHA * v0), jnp.tanh(ALPHA * v1),
                      jnp.tanh(ALPHA * v5))
        a0, a1, a5 = jnp.abs(v0), jnp.abs(v1), jnp.abs(v5)
        c0 = s * (t1 * t5) * _smin_pair(a1, a5)
        c1 = s * (t0 * t5) * _smin_pair(a0, a5)
        c5 = s * (t0 * t1) * _smin_pair(a0, a1)
        inc = c0 + jnp.roll(c1, 1, axis=0) + jnp.roll(c5, 5, axis=0)
        lv = inc + (1.0 - g) * p + g * lv
        v0 = lv - c0
        v1 = jnp.roll(lv, -1, axis=0) - c1
        v5 = jnp.roll(lv, -5, axis=0) - c5
    out_ref[...] = lv


def _tc_forward(syn_t, prior_llr, gamma, block, interpret=False):
    batch_tc = syn_t.shape[1]
    assert batch_tc % block == 0, (batch_tc, block)
    return pl.pallas_call(
        _tc_body,
        grid=(batch_tc // block,),
        in_specs=[
            pl.BlockSpec((M, block), lambda i: (0, i)),
            pl.BlockSpec((M, 1), lambda i: (0, 0)),
            pl.BlockSpec((M, 1), lambda i: (0, 0)),
        ],
        out_specs=pl.BlockSpec((M, block), lambda i: (0, i)),
        out_shape=jax.ShapeDtypeStruct((M, batch_tc), jnp.float32),
        compiler_params=pltpu.CompilerParams(
            dimension_semantics=("parallel",)),
        interpret=interpret,
    )(syn_t, prior_llr.reshape(M, 1), gamma.reshape(M, 1))


def _sc_forward(syn_t, prior_llr, gamma):
    batch = syn_t.shape[1]
    assert batch % (NW * LANES) == 0, batch
    bpw = batch // NW

    syn_blocks = syn_t.reshape(M, NW, bpw).transpose(1, 0, 2)
    pr16 = jnp.broadcast_to(prior_llr[:, None], (M, LANES))
    gm16 = jnp.broadcast_to(gamma[:, None], (M, LANES))

    mesh = plsc.VectorSubcoreMesh(core_axis_name="c", subcore_axis_name="s")
    run = pl.kernel(
        _bp_body,
        out_type=jax.ShapeDtypeStruct((NW, M, bpw), jnp.float32),
        mesh=mesh,
        scratch_types=[pltpu.VMEM((M, bpw), jnp.float32)] * 8
                      + [pltpu.VMEM((M, LANES), jnp.float32)] * 2,
    )
    out_blocks = run(syn_blocks, pr16, gm16)
    return out_blocks.transpose(1, 0, 2).reshape(M, batch)


SC_BATCH = 512
TC_BLOCK = 512


def kernel(syndromes, gamma, prior_llr):
    batch = syndromes.shape[0]
    syn_t = syndromes.astype(jnp.float32).T
    sc_b = 0
    out_sc = None
    if True:
        out_tc = _tc_forward(syn_t, prior_llr, gamma, TC_BLOCK)
        return out_tc.T
    if sc_b < batch:
        out_tc = _tc_forward(syn_t[:, sc_b:], prior_llr, gamma, TC_BLOCK)
        out = jnp.concatenate([out_sc, out_tc], axis=1)
    else:
        out = out_sc
    return out.T
